# Initial kernel scaffold; baseline (speedup 1.0000x reference)
import jax
import jax.numpy as jnp
from jax.experimental import pallas as pl


def kernel(x, batch, n_nodes, Omegas, Phis, Lambdas, Omegas_norm, Phis_norm, Lambdas_norm, gate_W1, gate_b1, gate_W2, gate_b2, feat_W1, feat_b1, feat_W2, feat_b2):
    raise NotImplementedError("write your pallas kernel here")


# speedup vs baseline: 3.1695x; 3.1695x over previous
import functools

import jax
import jax.numpy as jnp
from jax.experimental import pallas as pl
from jax.experimental.pallas import tpu as pltpu


def _body(x_ref, b_ref, W1_ref, b1_ref, gW2_ref, gb2_ref, fW2_ref, fb2_ref,
          out_ref, m_ref, d_ref, S_ref, *, R, P, Bn, DH, DE):
    i = pl.program_id(0)

    @pl.when(i == 0)
    def _init():
        m_ref[...] = jnp.full((P, Bn), -1e30, jnp.float32)
        d_ref[...] = jnp.zeros((P, Bn), jnp.float32)
        S_ref[...] = jnp.zeros((P, Bn, DE), jnp.float32)

    xb = x_ref[...]
    h = jax.lax.dot_general(xb, W1_ref[...], (((1,), (0,)), ((), ())),
                            preferred_element_type=jnp.float32)
    h = jnp.maximum(h + b1_ref[...], 0.0)
    g2 = jax.lax.dot_general(h[:, :P * DH], gW2_ref[...],
                             (((1,), (0,)), ((), ())),
                             preferred_element_type=jnp.float32)
    g2 = g2 + gb2_ref[...]

    bb = b_ref[...]
    seg_ids = jax.lax.broadcasted_iota(jnp.int32, (R, Bn), 1)
    O = bb == seg_ids

    for k in range(P):
        fk = jax.lax.dot_general(h[:, (P + k) * DH:(P + k + 1) * DH],
                                 fW2_ref[k], (((1,), (0,)), ((), ())),
                                 preferred_element_type=jnp.float32)
        fk = fk + fb2_ref[k]
        gk = g2[:, k:k + 1]
        masked = jnp.where(O, gk, -1e30)
        bmax = jnp.max(masked, axis=0, keepdims=True)
        m_old = m_ref[k:k + 1, :]
        m_new = jnp.maximum(m_old, bmax)
        scale = jnp.exp(m_old - m_new)
        E = jnp.where(O, jnp.exp(gk - m_new), 0.0)
        d_ref[k:k + 1, :] = (d_ref[k:k + 1, :] * scale
                             + jnp.sum(E, axis=0, keepdims=True))
        S_ref[k] = (S_ref[k] * jnp.transpose(scale)
                    + jax.lax.dot_general(E, fk, (((0,), (0,)), ((), ())),
                                          preferred_element_type=jnp.float32))
        m_ref[k:k + 1, :] = m_new

    @pl.when(i == pl.num_programs(0) - 1)
    def _finish():
        for k in range(P):
            dT = jnp.transpose(d_ref[k:k + 1, :])
            out_ref[k] = jnp.where(dT > 0.0, S_ref[k] / dT, 0.0)


def kernel(x, batch, n_nodes, Omegas, Phis, Lambdas, Omegas_norm, Phis_norm,
           Lambdas_norm, gate_W1, gate_b1, gate_W2, gate_b2, feat_W1, feat_b1,
           feat_W2, feat_b2):
    N, FD = x.shape
    Bn = n_nodes.shape[0]
    P, _, DH = gate_W1.shape
    DE = feat_W2.shape[2]
    R = 1000
    assert N % R == 0

    W1all = jnp.concatenate(
        [gate_W1[k] for k in range(P)] + [feat_W1[k] for k in range(P)], axis=1)
    b1all = jnp.concatenate(
        [gate_b1[k] for k in range(P)] + [feat_b1[k] for k in range(P)])[None, :]
    gW2c = jnp.zeros((P * DH, P), jnp.float32)
    for k in range(P):
        gW2c = gW2c.at[k * DH:(k + 1) * DH, k].set(gate_W2[k, :, 0])
    gb2c = gate_b2[:, 0][None, :]
    fb2r = feat_b2[:, None, :]
    batch2 = batch.astype(jnp.int32).reshape(N, 1)

    body = functools.partial(_body, R=R, P=P, Bn=Bn, DH=DH, DE=DE)
    pools = pl.pallas_call(
        body,
        grid=(N // R,),
        in_specs=[
            pl.BlockSpec((R, FD), lambda i: (i, 0)),
            pl.BlockSpec((R, 1), lambda i: (i, 0)),
            pl.BlockSpec((FD, 2 * P * DH), lambda i: (0, 0)),
            pl.BlockSpec((1, 2 * P * DH), lambda i: (0, 0)),
            pl.BlockSpec((P * DH, P), lambda i: (0, 0)),
            pl.BlockSpec((1, P), lambda i: (0, 0)),
            pl.BlockSpec((P, DH, DE), lambda i: (0, 0, 0)),
            pl.BlockSpec((P, 1, DE), lambda i: (0, 0, 0)),
        ],
        out_specs=pl.BlockSpec((P, Bn, DE), lambda i: (0, 0, 0)),
        out_shape=jax.ShapeDtypeStruct((P, Bn, DE), jnp.float32),
        scratch_shapes=[
            pltpu.VMEM((P, Bn), jnp.float32),
            pltpu.VMEM((P, Bn), jnp.float32),
            pltpu.VMEM((P, Bn, DE), jnp.float32),
        ],
    )(x, batch2, W1all, b1all, gW2c, gb2c, feat_W2, fb2r)

    return jnp.concatenate(
        [pools[k] for k in range(P)]
        + [n_nodes, Omegas, Phis, Lambdas, Omegas_norm, Phis_norm,
           Lambdas_norm], axis=1)


# speedup vs baseline: 3.5103x; 1.1075x over previous
import functools

import jax
import jax.numpy as jnp
from jax.experimental import pallas as pl
from jax.experimental.pallas import tpu as pltpu


def _body(x_ref, b_ref, W1_ref, b1_ref, gW2_ref, gb2_ref, fW2_ref, fb2_ref,
          out_ref, m_ref, d_ref, S_ref, *, R, P, Bn, DH, DE):
    i = pl.program_id(0)

    @pl.when(i == 0)
    def _init():
        m_ref[...] = jnp.full((P, Bn), -1e30, jnp.float32)
        d_ref[...] = jnp.zeros((P, Bn), jnp.float32)
        S_ref[...] = jnp.zeros((P, Bn, DE), jnp.float32)

    xb = x_ref[...]
    h = jax.lax.dot_general(xb, W1_ref[...], (((1,), (0,)), ((), ())),
                            preferred_element_type=jnp.float32)
    h = jnp.maximum(h + b1_ref[...], 0.0)
    g2 = jax.lax.dot_general(h[:, :P * DH], gW2_ref[...],
                             (((1,), (0,)), ((), ())),
                             preferred_element_type=jnp.float32)
    g2 = g2 + gb2_ref[...]

    bb = b_ref[...]
    seg_ids = jax.lax.broadcasted_iota(jnp.int32, (R, Bn), 1)
    O = bb == seg_ids

    for k in range(P):
        fk = jax.lax.dot_general(h[:, (P + k) * DH:(P + k + 1) * DH],
                                 fW2_ref[k], (((1,), (0,)), ((), ())),
                                 preferred_element_type=jnp.float32)
        fk = fk + fb2_ref[k]
        gk = g2[:, k:k + 1]
        masked = jnp.where(O, gk, -1e30)
        bmax = jnp.max(masked, axis=0, keepdims=True)
        m_old = m_ref[k:k + 1, :]
        m_new = jnp.maximum(m_old, bmax)
        scale = jnp.exp(m_old - m_new)
        E = jnp.where(O, jnp.exp(gk - m_new), 0.0)
        d_ref[k:k + 1, :] = (d_ref[k:k + 1, :] * scale
                             + jnp.sum(E, axis=0, keepdims=True))
        S_ref[k] = (S_ref[k] * jnp.transpose(scale)
                    + jax.lax.dot_general(E, fk, (((0,), (0,)), ((), ())),
                                          preferred_element_type=jnp.float32))
        m_ref[k:k + 1, :] = m_new

    @pl.when(i == pl.num_programs(0) - 1)
    def _finish():
        for k in range(P):
            dT = jnp.transpose(d_ref[k:k + 1, :])
            out_ref[k] = jnp.where(dT > 0.0, S_ref[k] / dT, 0.0)


def kernel(x, batch, n_nodes, Omegas, Phis, Lambdas, Omegas_norm, Phis_norm,
           Lambdas_norm, gate_W1, gate_b1, gate_W2, gate_b2, feat_W1, feat_b1,
           feat_W2, feat_b2):
    N, FD = x.shape
    Bn = n_nodes.shape[0]
    P, _, DH = gate_W1.shape
    DE = feat_W2.shape[2]
    R = 2000
    assert N % R == 0

    W1all = jnp.concatenate(
        [gate_W1[k] for k in range(P)] + [feat_W1[k] for k in range(P)], axis=1)
    b1all = jnp.concatenate(
        [gate_b1[k] for k in range(P)] + [feat_b1[k] for k in range(P)])[None, :]
    gW2c = jnp.zeros((P * DH, P), jnp.float32)
    for k in range(P):
        gW2c = gW2c.at[k * DH:(k + 1) * DH, k].set(gate_W2[k, :, 0])
    gb2c = gate_b2[:, 0][None, :]
    fb2r = feat_b2[:, None, :]
    batch2 = batch.astype(jnp.int32).reshape(N, 1)

    body = functools.partial(_body, R=R, P=P, Bn=Bn, DH=DH, DE=DE)
    pools = pl.pallas_call(
        body,
        grid=(N // R,),
        in_specs=[
            pl.BlockSpec((R, FD), lambda i: (i, 0)),
            pl.BlockSpec((R, 1), lambda i: (i, 0)),
            pl.BlockSpec((FD, 2 * P * DH), lambda i: (0, 0)),
            pl.BlockSpec((1, 2 * P * DH), lambda i: (0, 0)),
            pl.BlockSpec((P * DH, P), lambda i: (0, 0)),
            pl.BlockSpec((1, P), lambda i: (0, 0)),
            pl.BlockSpec((P, DH, DE), lambda i: (0, 0, 0)),
            pl.BlockSpec((P, 1, DE), lambda i: (0, 0, 0)),
        ],
        out_specs=pl.BlockSpec((P, Bn, DE), lambda i: (0, 0, 0)),
        out_shape=jax.ShapeDtypeStruct((P, Bn, DE), jnp.float32),
        scratch_shapes=[
            pltpu.VMEM((P, Bn), jnp.float32),
            pltpu.VMEM((P, Bn), jnp.float32),
            pltpu.VMEM((P, Bn, DE), jnp.float32),
        ],
    )(x, batch2, W1all, b1all, gW2c, gb2c, feat_W2, fb2r)

    return jnp.concatenate(
        [pools[k] for k in range(P)]
        + [n_nodes, Omegas, Phis, Lambdas, Omegas_norm, Phis_norm,
           Lambdas_norm], axis=1)


# speedup vs baseline: 3.6351x; 1.0356x over previous
import functools

import jax
import jax.numpy as jnp
from jax.experimental import pallas as pl
from jax.experimental.pallas import tpu as pltpu


def _body(x_ref, b_ref, W1_ref, b1_ref, gW2_ref, gb2_ref, fW2_ref, fb2_ref,
          out_ref, m_ref, d_ref, S_ref, *, R, P, Bn, DH, DE):
    i = pl.program_id(0)

    @pl.when(i == 0)
    def _init():
        m_ref[...] = jnp.full((P, Bn), -1e30, jnp.float32)
        d_ref[...] = jnp.zeros((P, Bn), jnp.float32)
        S_ref[...] = jnp.zeros((P, Bn, DE), jnp.float32)

    xb = x_ref[...]
    h = jax.lax.dot_general(xb, W1_ref[...], (((1,), (0,)), ((), ())),
                            preferred_element_type=jnp.float32)
    h = jnp.maximum(h + b1_ref[...], 0.0)
    g2 = jax.lax.dot_general(h[:, :P * DH], gW2_ref[...],
                             (((1,), (0,)), ((), ())),
                             preferred_element_type=jnp.float32)
    g2 = g2 + gb2_ref[...]

    bb = b_ref[...]
    seg_ids = jax.lax.broadcasted_iota(jnp.int32, (R, Bn), 1)
    O = bb == seg_ids

    for k in range(P):
        fk = jax.lax.dot_general(h[:, (P + k) * DH:(P + k + 1) * DH],
                                 fW2_ref[k], (((1,), (0,)), ((), ())),
                                 preferred_element_type=jnp.float32)
        fk = fk + fb2_ref[k]
        gk = g2[:, k:k + 1]
        masked = jnp.where(O, gk, -1e30)
        bmax = jnp.max(masked, axis=0, keepdims=True)
        m_old = m_ref[k:k + 1, :]
        m_new = jnp.maximum(m_old, bmax)
        scale = jnp.exp(m_old - m_new)
        E = jnp.where(O, jnp.exp(gk - m_new), 0.0)
        d_ref[k:k + 1, :] = (d_ref[k:k + 1, :] * scale
                             + jnp.sum(E, axis=0, keepdims=True))
        S_ref[k] = (S_ref[k] * jnp.transpose(scale)
                    + jax.lax.dot_general(E, fk, (((0,), (0,)), ((), ())),
                                          preferred_element_type=jnp.float32))
        m_ref[k:k + 1, :] = m_new

    @pl.when(i == pl.num_programs(0) - 1)
    def _finish():
        for k in range(P):
            dT = jnp.transpose(d_ref[k:k + 1, :])
            out_ref[k] = jnp.where(dT > 0.0, S_ref[k] / dT, 0.0)


def kernel(x, batch, n_nodes, Omegas, Phis, Lambdas, Omegas_norm, Phis_norm,
           Lambdas_norm, gate_W1, gate_b1, gate_W2, gate_b2, feat_W1, feat_b1,
           feat_W2, feat_b2):
    N, FD = x.shape
    Bn = n_nodes.shape[0]
    P, _, DH = gate_W1.shape
    DE = feat_W2.shape[2]
    R = 4000
    assert N % R == 0

    W1all = jnp.concatenate(
        [gate_W1[k] for k in range(P)] + [feat_W1[k] for k in range(P)], axis=1)
    b1all = jnp.concatenate(
        [gate_b1[k] for k in range(P)] + [feat_b1[k] for k in range(P)])[None, :]
    gW2c = jnp.zeros((P * DH, P), jnp.float32)
    for k in range(P):
        gW2c = gW2c.at[k * DH:(k + 1) * DH, k].set(gate_W2[k, :, 0])
    gb2c = gate_b2[:, 0][None, :]
    fb2r = feat_b2[:, None, :]
    batch2 = batch.astype(jnp.int32).reshape(N, 1)

    body = functools.partial(_body, R=R, P=P, Bn=Bn, DH=DH, DE=DE)
    pools = pl.pallas_call(
        body,
        grid=(N // R,),
        in_specs=[
            pl.BlockSpec((R, FD), lambda i: (i, 0)),
            pl.BlockSpec((R, 1), lambda i: (i, 0)),
            pl.BlockSpec((FD, 2 * P * DH), lambda i: (0, 0)),
            pl.BlockSpec((1, 2 * P * DH), lambda i: (0, 0)),
            pl.BlockSpec((P * DH, P), lambda i: (0, 0)),
            pl.BlockSpec((1, P), lambda i: (0, 0)),
            pl.BlockSpec((P, DH, DE), lambda i: (0, 0, 0)),
            pl.BlockSpec((P, 1, DE), lambda i: (0, 0, 0)),
        ],
        out_specs=pl.BlockSpec((P, Bn, DE), lambda i: (0, 0, 0)),
        out_shape=jax.ShapeDtypeStruct((P, Bn, DE), jnp.float32),
        scratch_shapes=[
            pltpu.VMEM((P, Bn), jnp.float32),
            pltpu.VMEM((P, Bn), jnp.float32),
            pltpu.VMEM((P, Bn, DE), jnp.float32),
        ],
    )(x, batch2, W1all, b1all, gW2c, gb2c, feat_W2, fb2r)

    return jnp.concatenate(
        [pools[k] for k in range(P)]
        + [n_nodes, Omegas, Phis, Lambdas, Omegas_norm, Phis_norm,
           Lambdas_norm], axis=1)
